# Initial kernel scaffold; baseline (speedup 1.0000x reference)
import jax
import jax.numpy as jnp
from jax.experimental import pallas as pl


def kernel(x, edge_index, batch, W1, att_src1, att_dst1, b1, W2, att_src2, att_dst2, b2, Wf1, bf1, Wf2, bf2):
    raise NotImplementedError("write your pallas kernel here")


# speedup vs baseline: 15.6929x; 15.6929x over previous
import functools

import jax
import jax.numpy as jnp
from jax import lax
from jax.experimental import pallas as pl
from jax.experimental.pallas import tpu as pltpu
from jax.experimental.pallas import tpu_sc as plsc

N = 10000
E2 = 640000
D = 128
HEADS = 4
H1 = 128
H2 = 64
NG = 64
NCLS = 10

NPT = 80
NTASK = 125
SCE = 256
SUB = 32
NEG_INF = float("-inf")


def _mm_att(x, w, a_s, a_d, act, bias):
    rows, din = x.shape
    dout = w.shape[1]
    ns = a_s.shape[1]
    blk = rows // 10

    def body(x_ref, w_ref, as_ref, ad_ref, b_ref, f_ref, ad_out_ref):
        xi = x_ref[...]
        if act:
            xi = jnp.maximum(xi + b_ref[...], 0.0)
        h = jnp.dot(xi, w_ref[...], preferred_element_type=jnp.float32)
        s = jnp.dot(h, as_ref[...], preferred_element_type=jnp.float32)
        f_ref[...] = jnp.concatenate([h, s], axis=1)
        ad_out_ref[...] = jnp.dot(h, ad_ref[...], preferred_element_type=jnp.float32)

    return pl.pallas_call(
        body,
        grid=(10,),
        in_specs=[
            pl.BlockSpec((blk, din), lambda i: (i, 0)),
            pl.BlockSpec((din, dout), lambda i: (0, 0)),
            pl.BlockSpec((dout, ns), lambda i: (0, 0)),
            pl.BlockSpec((dout, 128), lambda i: (0, 0)),
            pl.BlockSpec((1, din), lambda i: (0, 0)),
        ],
        out_specs=[
            pl.BlockSpec((blk, dout + ns), lambda i: (i, 0)),
            pl.BlockSpec((blk, 128), lambda i: (i, 0)),
        ],
        out_shape=[
            jax.ShapeDtypeStruct((rows, dout + ns), jnp.float32),
            jax.ShapeDtypeStruct((rows, 128), jnp.float32),
        ],
    )(x, w, a_s, a_d, bias)


def _head_body(h_ref, batch_ref, b2_ref, wf1_ref, bf1_ref, wf2_ref, bf2_ref,
               out_ref, g_ref):
    h = h_ref[...] + b2_ref[...]
    batch = batch_ref[...]

    def pool(g, _):
        m = jnp.max(jnp.where(batch == g, h, NEG_INF), axis=0, keepdims=True)
        g_ref[pl.ds(g, 1), :] = m
        return 0

    lax.fori_loop(0, NG, pool, 0)
    g = g_ref[...]
    a1 = jnp.maximum(jnp.dot(g, wf1_ref[...], preferred_element_type=jnp.float32)
                     + bf1_ref[...], 0.0)
    z = jnp.dot(a1, wf2_ref[...], preferred_element_type=jnp.float32) + bf2_ref[...]
    zm = jnp.max(z, axis=1, keepdims=True)
    out_ref[...] = (z - zm) - jnp.log(jnp.sum(jnp.exp(z - zm), axis=1, keepdims=True))


def _head(h, batch2d, b2, wf1, bf1, wf2, bf2):
    return pl.pallas_call(
        _head_body,
        out_shape=jax.ShapeDtypeStruct((NG, NCLS), jnp.float32),
        scratch_shapes=[pltpu.VMEM((NG, H2), jnp.float32)],
    )(h, batch2d, b2, wf1, bf1, wf2, bf2)


def _make_mp(C, H, FCOLS, FOFF):
    CPH = C // H
    NCH = C // 16
    mesh = plsc.VectorSubcoreMesh(core_axis_name="c", subcore_axis_name="s")

    def _sload(ref, idx):
        return ref[pl.ds(idx, 16)][0]

    @functools.partial(
        pl.kernel,
        out_type=jax.ShapeDtypeStruct((N, C), jnp.float32),
        mesh=mesh,
        compiler_params=pltpu.CompilerParams(needs_layout_passes=False),
        scratch_types=[
            pltpu.VMEM((152,), jnp.int32),
            pltpu.VMEM((SCE,), jnp.int32),
            pltpu.VMEM((SCE,), jnp.int32),
            pltpu.VMEM((SCE + 16,), jnp.int32),
            pltpu.VMEM((SUB, FCOLS), jnp.float32),
            pltpu.VMEM((NPT, C), jnp.float32),
            pltpu.VMEM((NPT * H + 16,), jnp.float32),
            pltpu.VMEM((NPT, 128), jnp.float32),
        ],
    )
    def mp(feat, adst_t, ew, tb, out, tbv, ewv, srcv, dlocv, hbuf, accv, dv,
           adl):
        wid = lax.axis_index("s") * 2 + lax.axis_index("c")
        pltpu.sync_copy(tb, tbv)
        zero16 = jnp.zeros((16,), jnp.float32)
        lanes = lax.iota(jnp.int32, 16)
        lmask = lanes < H

        def _round(r, _):
            t = r * 32 + wid
            v0 = t * NPT

            @pl.when(t < NTASK)
            def _task():
                pltpu.sync_copy(adst_t.at[pl.ds(pl.multiple_of(v0, 8), NPT)],
                                adl)

                def _z(i, _):
                    dv[pl.ds(i * 16, 16)] = zero16
                    return 0
                lax.fori_loop(0, NPT * H // 16, _z, 0)

                def _za(v, _):
                    for c in range(NCH):
                        accv[v, pl.ds(c * 16, 16)] = zero16
                    return 0
                lax.fori_loop(0, NPT, _za, 0)

                e0 = _sload(tbv, t)
                e1 = _sload(tbv, t + 1)
                sc0 = e0 // SCE
                sc1 = (e1 + SCE - 1) // SCE

                def _chunk(kc, _):
                    ca = pl.multiple_of(kc * SCE, SCE)
                    pltpu.sync_copy(ew.at[pl.ds(ca, SCE)], ewv)
                    for i in range(SCE // 16):
                        w16 = ewv[pl.ds(i * 16, 16)]
                        srcv[pl.ds(i * 16, 16)] = w16 & 0x3FFF
                        dl = ((w16 >> 14) & 0x3FFF) - v0
                        dlocv[pl.ds(i * 16, 16)] = jnp.clip(dl, 0, NPT - 1)
                    for sub in range(SCE // SUB):
                        pltpu.sync_copy(
                            feat.at[srcv.at[pl.ds(sub * SUB, SUB)]], hbuf)

                        def _edge(e, _):
                            eg = sub * SUB + e
                            ewe = _sload(ewv, eg)
                            dl = _sload(dlocv, eg)
                            abs_e = ca + eg
                            ok = ((abs_e >= e0) & (abs_e < e1)
                                  & ((ewe >> 28) == 0))
                            asrc = hbuf[e, pl.ds(FOFF, 16)]
                            al = asrc + adl[dl, pl.ds(0, 16)]
                            al = jnp.where(al >= 0.0, al, 0.2 * al)
                            w = jnp.where(ok & lmask, jnp.exp(al), 0.0)
                            plsc.addupdate_scatter(dv, [dl * H + lanes], w)
                            for c in range(NCH):
                                f16 = hbuf[e, pl.ds(c * 16, 16)]
                                plsc.addupdate(
                                    accv.at[dl, pl.ds(c * 16, 16)],
                                    w[c * 16 // CPH] * f16)
                            return 0
                        lax.fori_loop(0, SUB, _edge, 0)
                    return 0

                lax.fori_loop(sc0, sc1, _chunk, 0)

                def _inv(i, _):
                    d16 = dv[pl.ds(i * 16, 16)]
                    dv[pl.ds(i * 16, 16)] = 1.0 / (d16 + 1e-16)
                    return 0
                lax.fori_loop(0, NPT * H // 16, _inv, 0)

                def _fin(v, _):
                    for h in range(H):
                        iv = _sload(dv, v * H + h)
                        for c in range(CPH // 16):
                            cc = h * CPH + c * 16
                            accv[v, pl.ds(cc, 16)] = accv[v, pl.ds(cc, 16)] * iv
                    return 0
                lax.fori_loop(0, NPT, _fin, 0)
                pltpu.sync_copy(accv,
                                out.at[pl.ds(pl.multiple_of(v0, 8), NPT)])
            return 0

        lax.fori_loop(0, 4, _round, 0)

    return mp


_mp1 = _make_mp(512, HEADS, 640, 512)
_mp2 = _make_mp(128, 1, 128, 64)


def kernel(x, edge_index, batch, W1, att_src1, att_dst1, b1,
           W2, att_src2, att_dst2, b2, Wf1, bf1, Wf2, bf2):
    row = jnp.concatenate([edge_index[0], edge_index[1]]).astype(jnp.int32)
    col = jnp.concatenate([edge_index[1], edge_index[0]]).astype(jnp.int32)
    key = jnp.sort(col * N + row)
    dup = jnp.concatenate([jnp.zeros((1,), jnp.bool_), key[1:] == key[:-1]])
    src = key % N
    dst = key // N
    ew = (src | (dst << 14) | (dup.astype(jnp.int32) << 28)).astype(jnp.int32)
    bounds = jnp.clip(jnp.arange(152, dtype=jnp.int32) * NPT, 0, N)
    tb = jnp.searchsorted(key, bounds * N, side="left").astype(jnp.int32)

    a1s = jnp.zeros((HEADS * H1, 128), jnp.float32)
    a1d = jnp.zeros((HEADS * H1, 128), jnp.float32)
    for h in range(HEADS):
        a1s = a1s.at[h * H1:(h + 1) * H1, h].set(att_src1[h])
        a1d = a1d.at[h * H1:(h + 1) * H1, h].set(att_dst1[h])
    a2s = jnp.zeros((H2, 64), jnp.float32).at[:, 0].set(att_src2[0])
    a2d = jnp.zeros((H2, 128), jnp.float32).at[:, 0].set(att_dst2[0])

    zerod = jnp.zeros((1, D), jnp.float32)

    f1, ad1 = _mm_att(x, W1, a1s, a1d, False, zerod)
    g1 = _mp1(f1, ad1, ew, tb)
    f2, ad2 = _mm_att(g1, W2, a2s, a2d, True, b1.reshape(1, -1))
    g2 = _mp2(f2, ad2, ew, tb)
    return _head(g2[:, :H2], batch.reshape(-1, 1), b2.reshape(1, -1),
                 Wf1, bf1.reshape(1, -1), Wf2, bf2.reshape(1, -1))


# speedup vs baseline: 19.9128x; 1.2689x over previous
import functools

import jax
import jax.numpy as jnp
from jax import lax
from jax.experimental import pallas as pl
from jax.experimental.pallas import tpu as pltpu
from jax.experimental.pallas import tpu_sc as plsc

N = 10000
E2 = 640000
D = 128
HEADS = 4
H1 = 128
H2 = 64
NG = 64
NCLS = 10

NPT = 80
NTASK = 125
SCE = 256
NEG_INF = float("-inf")


def _mm_att(x, w, a_s, a_d, act, bias):
    rows, din = x.shape
    dout = w.shape[1]
    ns = a_s.shape[1]
    blk = rows // 10

    def body(x_ref, w_ref, as_ref, ad_ref, b_ref, f_ref, ad_out_ref):
        xi = x_ref[...]
        if act:
            xi = jnp.maximum(xi + b_ref[...], 0.0)
        h = jnp.dot(xi, w_ref[...], preferred_element_type=jnp.float32)
        s = jnp.dot(h, as_ref[...], preferred_element_type=jnp.float32)
        f_ref[...] = jnp.concatenate([h, s], axis=1)
        ad_out_ref[...] = jnp.dot(h, ad_ref[...], preferred_element_type=jnp.float32)

    return pl.pallas_call(
        body,
        grid=(10,),
        in_specs=[
            pl.BlockSpec((blk, din), lambda i: (i, 0)),
            pl.BlockSpec((din, dout), lambda i: (0, 0)),
            pl.BlockSpec((dout, ns), lambda i: (0, 0)),
            pl.BlockSpec((dout, 128), lambda i: (0, 0)),
            pl.BlockSpec((1, din), lambda i: (0, 0)),
        ],
        out_specs=[
            pl.BlockSpec((blk, dout + ns), lambda i: (i, 0)),
            pl.BlockSpec((blk, 128), lambda i: (i, 0)),
        ],
        out_shape=[
            jax.ShapeDtypeStruct((rows, dout + ns), jnp.float32),
            jax.ShapeDtypeStruct((rows, 128), jnp.float32),
        ],
    )(x, w, a_s, a_d, bias)


def _head_body(h_ref, batch_ref, b2_ref, wf1_ref, bf1_ref, wf2_ref, bf2_ref,
               out_ref, g_ref):
    h = h_ref[...] + b2_ref[...]
    batch = batch_ref[...]

    def pool(g, _):
        m = jnp.max(jnp.where(batch == g, h, NEG_INF), axis=0, keepdims=True)
        g_ref[pl.ds(g, 1), :] = m
        return 0

    lax.fori_loop(0, NG, pool, 0)
    g = g_ref[...]
    a1 = jnp.maximum(jnp.dot(g, wf1_ref[...], preferred_element_type=jnp.float32)
                     + bf1_ref[...], 0.0)
    z = jnp.dot(a1, wf2_ref[...], preferred_element_type=jnp.float32) + bf2_ref[...]
    zm = jnp.max(z, axis=1, keepdims=True)
    out_ref[...] = (z - zm) - jnp.log(jnp.sum(jnp.exp(z - zm), axis=1, keepdims=True))


def _head(h, batch2d, b2, wf1, bf1, wf2, bf2):
    return pl.pallas_call(
        _head_body,
        out_shape=jax.ShapeDtypeStruct((NG, NCLS), jnp.float32),
        scratch_shapes=[pltpu.VMEM((NG, H2), jnp.float32)],
    )(h, batch2d, b2, wf1, bf1, wf2, bf2)


def _make_mp(C, H, FCOLS, FOFF, CACC, SUB):
    CPH = CACC // H
    NCH = CACC // 16
    NSUB = SCE // SUB
    mesh = plsc.VectorSubcoreMesh(core_axis_name="c", subcore_axis_name="s")

    def _sload(ref, idx):
        return ref[pl.ds(idx, 16)][0]

    @functools.partial(
        pl.kernel,
        out_type=jax.ShapeDtypeStruct((N, C), jnp.float32),
        mesh=mesh,
        compiler_params=pltpu.CompilerParams(needs_layout_passes=False),
        scratch_types=[
            pltpu.VMEM((152,), jnp.int32),
            pltpu.VMEM((SCE,), jnp.int32),
            pltpu.VMEM((SCE,), jnp.int32),
            pltpu.VMEM((SCE + 16,), jnp.int32),
            pltpu.VMEM((2, SUB, FCOLS), jnp.float32),
            pltpu.VMEM((NPT, C), jnp.float32),
            pltpu.VMEM((NPT * H + 16,), jnp.float32),
            pltpu.VMEM((NPT, 128), jnp.float32),
            pltpu.SemaphoreType.DMA((2,)),
        ],
    )
    def mp(feat, adst_t, ew, tb, out, tbv, ewv, srcv, dlocv, hbuf, accv, dv,
           adl, sem):
        wid = lax.axis_index("s") * 2 + lax.axis_index("c")
        pltpu.sync_copy(tb, tbv)
        zero16 = jnp.zeros((16,), jnp.float32)
        lanes = lax.iota(jnp.int32, 16)
        lmask = lanes < H

        def _round(r, _):
            t = r * 32 + wid
            v0 = t * NPT

            @pl.when(t < NTASK)
            def _task():
                pltpu.sync_copy(adst_t.at[pl.ds(pl.multiple_of(v0, 8), NPT)],
                                adl)

                def _z(i, _):
                    dv[pl.ds(i * 16, 16)] = zero16
                    return 0
                lax.fori_loop(0, NPT * H // 16, _z, 0)

                def _za(v, _):
                    for c in range(NCH):
                        accv[v, pl.ds(c * 16, 16)] = zero16
                    return 0
                lax.fori_loop(0, NPT, _za, 0)

                e0 = _sload(tbv, t)
                e1 = _sload(tbv, t + 1)
                sc0 = e0 // SCE
                sc1 = (e1 + SCE - 1) // SCE

                def _chunk(kc, _):
                    ca = pl.multiple_of(kc * SCE, SCE)
                    pltpu.sync_copy(ew.at[pl.ds(ca, SCE)], ewv)
                    for i in range(SCE // 16):
                        w16 = ewv[pl.ds(i * 16, 16)]
                        srcv[pl.ds(i * 16, 16)] = w16 & 0x3FFF
                        dl = ((w16 >> 14) & 0x3FFF) - v0
                        dlocv[pl.ds(i * 16, 16)] = jnp.clip(dl, 0, NPT - 1)

                    def _gather(sub, b):
                        return pltpu.async_copy(
                            feat.at[srcv.at[pl.ds(sub * SUB, SUB)]],
                            hbuf.at[b], sem.at[b])

                    descs = [None, None]
                    descs[0] = _gather(0, 0)
                    for sub in range(NSUB):
                        b = sub % 2
                        descs[b].wait()
                        if sub + 1 < NSUB:
                            descs[1 - b] = _gather(sub + 1, 1 - b)

                        def _edge(e, _):
                            eg = sub * SUB + e
                            ewe = _sload(ewv, eg)
                            dl = _sload(dlocv, eg)
                            abs_e = ca + eg
                            ok = ((abs_e >= e0) & (abs_e < e1)
                                  & ((ewe >> 28) == 0))
                            asrc = hbuf[b, e, pl.ds(FOFF, 16)]
                            al = asrc + adl[dl, pl.ds(0, 16)]
                            al = jnp.where(al >= 0.0, al, 0.2 * al)
                            w = jnp.where(ok & lmask, jnp.exp(al), 0.0)
                            plsc.addupdate_scatter(dv, [dl * H + lanes], w)
                            for c in range(NCH):
                                f16 = hbuf[b, e, pl.ds(c * 16, 16)]
                                plsc.addupdate(
                                    accv.at[dl, pl.ds(c * 16, 16)],
                                    w[c * 16 // CPH] * f16)
                            return 0
                        lax.fori_loop(0, SUB, _edge, 0)
                    return 0

                lax.fori_loop(sc0, sc1, _chunk, 0)

                def _inv(i, _):
                    d16 = dv[pl.ds(i * 16, 16)]
                    dv[pl.ds(i * 16, 16)] = 1.0 / (d16 + 1e-16)
                    return 0
                lax.fori_loop(0, NPT * H // 16, _inv, 0)

                def _fin(v, _):
                    for h in range(H):
                        iv = _sload(dv, v * H + h)
                        for c in range(CPH // 16):
                            cc = h * CPH + c * 16
                            accv[v, pl.ds(cc, 16)] = accv[v, pl.ds(cc, 16)] * iv
                    return 0
                lax.fori_loop(0, NPT, _fin, 0)
                pltpu.sync_copy(accv,
                                out.at[pl.ds(pl.multiple_of(v0, 8), NPT)])
            return 0

        lax.fori_loop(0, 4, _round, 0)

    return mp


_mp1 = _make_mp(512, HEADS, 640, 512, 512, 32)
_mp2 = _make_mp(128, 1, 128, 64, 64, 128)


def kernel(x, edge_index, batch, W1, att_src1, att_dst1, b1,
           W2, att_src2, att_dst2, b2, Wf1, bf1, Wf2, bf2):
    row = jnp.concatenate([edge_index[0], edge_index[1]]).astype(jnp.int32)
    col = jnp.concatenate([edge_index[1], edge_index[0]]).astype(jnp.int32)
    key = jnp.sort(col * N + row)
    dup = jnp.concatenate([jnp.zeros((1,), jnp.bool_), key[1:] == key[:-1]])
    src = key % N
    dst = key // N
    ew = (src | (dst << 14) | (dup.astype(jnp.int32) << 28)).astype(jnp.int32)
    bounds = jnp.clip(jnp.arange(152, dtype=jnp.int32) * NPT, 0, N)
    tb = jnp.searchsorted(key, bounds * N, side="left").astype(jnp.int32)

    a1s = jnp.zeros((HEADS * H1, 128), jnp.float32)
    a1d = jnp.zeros((HEADS * H1, 128), jnp.float32)
    for h in range(HEADS):
        a1s = a1s.at[h * H1:(h + 1) * H1, h].set(att_src1[h])
        a1d = a1d.at[h * H1:(h + 1) * H1, h].set(att_dst1[h])
    a2s = jnp.zeros((H2, 64), jnp.float32).at[:, 0].set(att_src2[0])
    a2d = jnp.zeros((H2, 128), jnp.float32).at[:, 0].set(att_dst2[0])

    zerod = jnp.zeros((1, D), jnp.float32)

    f1, ad1 = _mm_att(x, W1, a1s, a1d, False, zerod)
    g1 = _mp1(f1, ad1, ew, tb)
    f2, ad2 = _mm_att(g1, W2, a2s, a2d, True, b1.reshape(1, -1))
    g2 = _mp2(f2, ad2, ew, tb)
    return _head(g2[:, :H2], batch.reshape(-1, 1), b2.reshape(1, -1),
                 Wf1, bf1.reshape(1, -1), Wf2, bf2.reshape(1, -1))


# speedup vs baseline: 21.5483x; 1.0821x over previous
import functools

import jax
import jax.numpy as jnp
from jax import lax
from jax.experimental import pallas as pl
from jax.experimental.pallas import tpu as pltpu
from jax.experimental.pallas import tpu_sc as plsc

N = 10000
E2 = 640000
D = 128
HEADS = 4
H1 = 128
H2 = 64
NG = 64
NCLS = 10

NPT = 80
NTASK = 125
SCE = 256
NEG_INF = float("-inf")


def _mm_att(x, w, a_s, a_d, act, bias):
    rows, din = x.shape
    dout = w.shape[1]
    ns = a_s.shape[1]
    blk = rows // 10

    def body(x_ref, w_ref, as_ref, ad_ref, b_ref, f_ref, ad_out_ref):
        xi = x_ref[...]
        if act:
            xi = jnp.maximum(xi + b_ref[...], 0.0)
        h = jnp.dot(xi, w_ref[...], preferred_element_type=jnp.float32)
        s = jnp.dot(h, as_ref[...], preferred_element_type=jnp.float32)
        f_ref[...] = jnp.concatenate([h, s], axis=1)
        ad_out_ref[...] = jnp.dot(h, ad_ref[...], preferred_element_type=jnp.float32)

    return pl.pallas_call(
        body,
        grid=(10,),
        in_specs=[
            pl.BlockSpec((blk, din), lambda i: (i, 0)),
            pl.BlockSpec((din, dout), lambda i: (0, 0)),
            pl.BlockSpec((dout, ns), lambda i: (0, 0)),
            pl.BlockSpec((dout, 128), lambda i: (0, 0)),
            pl.BlockSpec((1, din), lambda i: (0, 0)),
        ],
        out_specs=[
            pl.BlockSpec((blk, dout + ns), lambda i: (i, 0)),
            pl.BlockSpec((blk, 128), lambda i: (i, 0)),
        ],
        out_shape=[
            jax.ShapeDtypeStruct((rows, dout + ns), jnp.float32),
            jax.ShapeDtypeStruct((rows, 128), jnp.float32),
        ],
    )(x, w, a_s, a_d, bias)


def _head_body(h_ref, batch_ref, b2_ref, wf1_ref, bf1_ref, wf2_ref, bf2_ref,
               out_ref, g_ref):
    h = h_ref[...] + b2_ref[...]
    batch = batch_ref[...]

    def pool(g, _):
        m = jnp.max(jnp.where(batch == g, h, NEG_INF), axis=0, keepdims=True)
        g_ref[pl.ds(g, 1), :] = m
        return 0

    lax.fori_loop(0, NG, pool, 0)
    g = g_ref[...]
    a1 = jnp.maximum(jnp.dot(g, wf1_ref[...], preferred_element_type=jnp.float32)
                     + bf1_ref[...], 0.0)
    z = jnp.dot(a1, wf2_ref[...], preferred_element_type=jnp.float32) + bf2_ref[...]
    zm = jnp.max(z, axis=1, keepdims=True)
    out_ref[...] = (z - zm) - jnp.log(jnp.sum(jnp.exp(z - zm), axis=1, keepdims=True))


def _head(h, batch2d, b2, wf1, bf1, wf2, bf2):
    return pl.pallas_call(
        _head_body,
        out_shape=jax.ShapeDtypeStruct((NG, NCLS), jnp.float32),
        scratch_shapes=[pltpu.VMEM((NG, H2), jnp.float32)],
    )(h, batch2d, b2, wf1, bf1, wf2, bf2)


def _make_mp(C, H, FCOLS, FOFF, CACC, SUB):
    CPH = CACC // H
    NCH = CACC // 16
    NSUB = SCE // SUB
    mesh = plsc.VectorSubcoreMesh(core_axis_name="c", subcore_axis_name="s")

    def _sload(ref, idx):
        return ref[pl.ds(idx, 16)][0]

    @functools.partial(
        pl.kernel,
        out_type=jax.ShapeDtypeStruct((N, C), jnp.float32),
        mesh=mesh,
        compiler_params=pltpu.CompilerParams(needs_layout_passes=False),
        scratch_types=[
            pltpu.VMEM((152,), jnp.int32),
            pltpu.VMEM((SCE,), jnp.int32),
            pltpu.VMEM((SCE,), jnp.int32),
            pltpu.VMEM((SCE + 16,), jnp.int32),
            pltpu.VMEM((2, SUB, FCOLS), jnp.float32),
            pltpu.VMEM((NPT + 1, C), jnp.float32),
            pltpu.VMEM((NPT + 1, 16), jnp.float32),
            pltpu.VMEM((NPT + 1, 128), jnp.float32),
            pltpu.SemaphoreType.DMA((2,)),
        ],
    )
    def mp(feat, adst_t, ew, tb, out, tbv, ewv, srcv, dlocv, hbuf, accv, dacc,
           adl, sem):
        wid = lax.axis_index("s") * 2 + lax.axis_index("c")
        pltpu.sync_copy(tb, tbv)
        zero16 = jnp.zeros((16,), jnp.float32)
        lanes = lax.iota(jnp.int32, 16)

        def _round(r, _):
            t = r * 32 + wid
            v0 = t * NPT

            @pl.when(t < NTASK)
            def _task():
                pltpu.sync_copy(adst_t.at[pl.ds(pl.multiple_of(v0, 8), NPT)],
                                adl.at[pl.ds(0, NPT)])

                def _za(v, _):
                    dacc[v, pl.ds(0, 16)] = zero16
                    for c in range(NCH):
                        accv[v, pl.ds(c * 16, 16)] = zero16
                    return 0
                lax.fori_loop(0, NPT, _za, 0)

                e0 = _sload(tbv, t)
                e1 = _sload(tbv, t + 1)
                sc0 = e0 // SCE
                sc1 = (e1 + SCE - 1) // SCE

                def _chunk(kc, _):
                    ca = pl.multiple_of(kc * SCE, SCE)
                    pltpu.sync_copy(ew.at[pl.ds(ca, SCE)], ewv)
                    for i in range(SCE // 16):
                        w16 = ewv[pl.ds(i * 16, 16)]
                        srcv[pl.ds(i * 16, 16)] = w16 & 0x3FFF
                        abs16 = ca + i * 16 + lanes
                        valid = ((abs16 >= e0) & (abs16 < e1)
                                 & ((w16 >> 28) == 0))
                        dl = ((w16 >> 14) & 0x3FFF) - v0
                        dlocv[pl.ds(i * 16, 16)] = jnp.where(valid, dl, NPT)

                    def _gather(sub, b):
                        return pltpu.async_copy(
                            feat.at[srcv.at[pl.ds(sub * SUB, SUB)]],
                            hbuf.at[b], sem.at[b])

                    descs = [None, None]
                    descs[0] = _gather(0, 0)
                    for sub in range(NSUB):
                        b = sub % 2
                        descs[b].wait()
                        if sub + 1 < NSUB:
                            descs[1 - b] = _gather(sub + 1, 1 - b)

                        def _grp(g, _):
                            dl16 = dlocv[pl.ds(sub * SUB + g * 4, 16)]
                            for j in range(4):
                                e = g * 4 + j
                                dl = dl16[j]
                                asrc = hbuf[b, e, pl.ds(FOFF, 16)]
                                al = asrc + adl[dl, pl.ds(0, 16)]
                                al = jnp.where(al >= 0.0, al, 0.2 * al)
                                w = jnp.exp(al)
                                plsc.addupdate(dacc.at[dl, pl.ds(0, 16)], w)
                                for c in range(NCH):
                                    f16 = hbuf[b, e, pl.ds(c * 16, 16)]
                                    plsc.addupdate(
                                        accv.at[dl, pl.ds(c * 16, 16)],
                                        w[c * 16 // CPH] * f16)
                            return 0
                        lax.fori_loop(0, SUB // 4, _grp, 0)
                    return 0

                lax.fori_loop(sc0, sc1, _chunk, 0)

                def _fin(v, _):
                    inv = 1.0 / (dacc[v, pl.ds(0, 16)] + 1e-16)
                    for h in range(H):
                        iv = inv[h]
                        for c in range(CPH // 16):
                            cc = h * CPH + c * 16
                            accv[v, pl.ds(cc, 16)] = accv[v, pl.ds(cc, 16)] * iv
                    return 0
                lax.fori_loop(0, NPT, _fin, 0)
                pltpu.sync_copy(accv.at[pl.ds(0, NPT)],
                                out.at[pl.ds(pl.multiple_of(v0, 8), NPT)])
            return 0

        lax.fori_loop(0, 4, _round, 0)

    return mp


_mp1 = _make_mp(512, HEADS, 640, 512, 512, 32)
_mp2 = _make_mp(128, 1, 128, 64, 64, 128)


def kernel(x, edge_index, batch, W1, att_src1, att_dst1, b1,
           W2, att_src2, att_dst2, b2, Wf1, bf1, Wf2, bf2):
    row = jnp.concatenate([edge_index[0], edge_index[1]]).astype(jnp.int32)
    col = jnp.concatenate([edge_index[1], edge_index[0]]).astype(jnp.int32)
    key = jnp.sort(col * N + row)
    dup = jnp.concatenate([jnp.zeros((1,), jnp.bool_), key[1:] == key[:-1]])
    src = key % N
    dst = key // N
    ew = (src | (dst << 14) | (dup.astype(jnp.int32) << 28)).astype(jnp.int32)
    bounds = jnp.clip(jnp.arange(152, dtype=jnp.int32) * NPT, 0, N)
    tb = jnp.searchsorted(key, bounds * N, side="left").astype(jnp.int32)

    a1s = jnp.zeros((HEADS * H1, 128), jnp.float32)
    a1d = jnp.zeros((HEADS * H1, 128), jnp.float32)
    for h in range(HEADS):
        a1s = a1s.at[h * H1:(h + 1) * H1, h].set(att_src1[h])
        a1d = a1d.at[h * H1:(h + 1) * H1, h].set(att_dst1[h])
    a2s = jnp.zeros((H2, 64), jnp.float32).at[:, 0].set(att_src2[0])
    a2d = jnp.zeros((H2, 128), jnp.float32).at[:, 0].set(att_dst2[0])

    zerod = jnp.zeros((1, D), jnp.float32)

    f1, ad1 = _mm_att(x, W1, a1s, a1d, False, zerod)
    g1 = _mp1(f1, ad1, ew, tb)
    f2, ad2 = _mm_att(g1, W2, a2s, a2d, True, b1.reshape(1, -1))
    g2 = _mp2(f2, ad2, ew, tb)
    return _head(g2[:, :H2], batch.reshape(-1, 1), b2.reshape(1, -1),
                 Wf1, bf1.reshape(1, -1), Wf2, bf2.reshape(1, -1))


# speedup vs baseline: 21.5629x; 1.0007x over previous
import functools

import jax
import jax.numpy as jnp
from jax import lax
from jax.experimental import pallas as pl
from jax.experimental.pallas import tpu as pltpu
from jax.experimental.pallas import tpu_sc as plsc

N = 10000
E2 = 640000
D = 128
HEADS = 4
H1 = 128
H2 = 64
NG = 64
NCLS = 10

NPT = 80
NTASK = 125
SCE = 256
NEG_INF = float("-inf")


def _mm_att(x, w, a_s, a_d, act, bias):
    rows, din = x.shape
    dout = w.shape[1]
    ns = a_s.shape[1]
    blk = rows // 10

    def body(x_ref, w_ref, as_ref, ad_ref, b_ref, f_ref, ad_out_ref):
        xi = x_ref[...]
        if act:
            xi = jnp.maximum(xi + b_ref[...], 0.0)
        h = jnp.dot(xi, w_ref[...], preferred_element_type=jnp.float32)
        s = jnp.dot(h, as_ref[...], preferred_element_type=jnp.float32)
        f_ref[...] = jnp.concatenate([h, s], axis=1)
        ad_out_ref[...] = jnp.dot(h, ad_ref[...], preferred_element_type=jnp.float32)

    return pl.pallas_call(
        body,
        grid=(10,),
        in_specs=[
            pl.BlockSpec((blk, din), lambda i: (i, 0)),
            pl.BlockSpec((din, dout), lambda i: (0, 0)),
            pl.BlockSpec((dout, ns), lambda i: (0, 0)),
            pl.BlockSpec((dout, 128), lambda i: (0, 0)),
            pl.BlockSpec((1, din), lambda i: (0, 0)),
        ],
        out_specs=[
            pl.BlockSpec((blk, dout + ns), lambda i: (i, 0)),
            pl.BlockSpec((blk, 128), lambda i: (i, 0)),
        ],
        out_shape=[
            jax.ShapeDtypeStruct((rows, dout + ns), jnp.float32),
            jax.ShapeDtypeStruct((rows, 128), jnp.float32),
        ],
    )(x, w, a_s, a_d, bias)


def _head_body(h_ref, batch_ref, b2_ref, wf1_ref, bf1_ref, wf2_ref, bf2_ref,
               out_ref, g_ref):
    h = h_ref[...] + b2_ref[...]
    batch = batch_ref[...]

    def pool(g, _):
        m = jnp.max(jnp.where(batch == g, h, NEG_INF), axis=0, keepdims=True)
        g_ref[pl.ds(g, 1), :] = m
        return 0

    lax.fori_loop(0, NG, pool, 0)
    g = g_ref[...]
    a1 = jnp.maximum(jnp.dot(g, wf1_ref[...], preferred_element_type=jnp.float32)
                     + bf1_ref[...], 0.0)
    z = jnp.dot(a1, wf2_ref[...], preferred_element_type=jnp.float32) + bf2_ref[...]
    zm = jnp.max(z, axis=1, keepdims=True)
    out_ref[...] = (z - zm) - jnp.log(jnp.sum(jnp.exp(z - zm), axis=1, keepdims=True))


def _head(h, batch2d, b2, wf1, bf1, wf2, bf2):
    return pl.pallas_call(
        _head_body,
        out_shape=jax.ShapeDtypeStruct((NG, NCLS), jnp.float32),
        scratch_shapes=[pltpu.VMEM((NG, H2), jnp.float32)],
    )(h, batch2d, b2, wf1, bf1, wf2, bf2)


def _make_mp(C, H, FCOLS, FOFF, CACC, SUB):
    CPH = CACC // H
    NCH = CACC // 16
    NSUB = SCE // SUB
    mesh = plsc.VectorSubcoreMesh(core_axis_name="c", subcore_axis_name="s")

    def _sload(ref, idx):
        return ref[pl.ds(idx, 16)][0]

    @functools.partial(
        pl.kernel,
        out_type=jax.ShapeDtypeStruct((N, C), jnp.float32),
        mesh=mesh,
        compiler_params=pltpu.CompilerParams(needs_layout_passes=False),
        scratch_types=[
            pltpu.VMEM((152,), jnp.int32),
            pltpu.VMEM((SCE,), jnp.int32),
            pltpu.VMEM((SCE,), jnp.int32),
            pltpu.VMEM((SCE + 16,), jnp.int32),
            pltpu.VMEM((2, SUB, FCOLS), jnp.float32),
            pltpu.VMEM((NPT + 1, C), jnp.float32),
            pltpu.VMEM((NPT + 1, 16), jnp.float32),
            pltpu.VMEM((NPT + 1, 128), jnp.float32),
            pltpu.SemaphoreType.DMA((2,)),
        ],
    )
    def mp(feat, adst_t, ew, tb, out, tbv, ewv, srcv, dlocv, hbuf, accv, dacc,
           adl, sem):
        wid = lax.axis_index("s") * 2 + lax.axis_index("c")
        pltpu.sync_copy(tb, tbv)
        zero16 = jnp.zeros((16,), jnp.float32)
        lanes = lax.iota(jnp.int32, 16)

        def _round(r, _):
            t = r * 32 + wid
            v0 = t * NPT

            @pl.when(t < NTASK)
            def _task():
                pltpu.sync_copy(adst_t.at[pl.ds(pl.multiple_of(v0, 8), NPT)],
                                adl.at[pl.ds(0, NPT)])

                def _za(v, _):
                    dacc[v, pl.ds(0, 16)] = zero16
                    for c in range(NCH):
                        accv[v, pl.ds(c * 16, 16)] = zero16
                    return 0
                lax.fori_loop(0, NPT, _za, 0)

                e0 = _sload(tbv, t)
                e1 = _sload(tbv, t + 1)
                sc0 = e0 // SCE
                sc1 = (e1 + SCE - 1) // SCE

                def _chunk(kc, _):
                    ca = pl.multiple_of(kc * SCE, SCE)
                    pltpu.sync_copy(ew.at[pl.ds(ca, SCE)], ewv)
                    for i in range(SCE // 16):
                        w16 = ewv[pl.ds(i * 16, 16)]
                        srcv[pl.ds(i * 16, 16)] = w16 & 0x3FFF
                        abs16 = ca + i * 16 + lanes
                        valid = ((abs16 >= e0) & (abs16 < e1)
                                 & ((w16 >> 28) == 0))
                        dl = ((w16 >> 14) & 0x3FFF) - v0
                        dlocv[pl.ds(i * 16, 16)] = jnp.where(valid, dl, NPT)

                    def _gather(sub, b):
                        return pltpu.async_copy(
                            feat.at[srcv.at[pl.ds(sub * SUB, SUB)]],
                            hbuf.at[b], sem.at[b])

                    descs = [None, None]
                    descs[0] = _gather(0, 0)
                    for sub in range(NSUB):
                        b = sub % 2
                        descs[b].wait()
                        if sub + 1 < NSUB:
                            descs[1 - b] = _gather(sub + 1, 1 - b)

                        def _grp(g, _):
                            dl16 = dlocv[pl.ds(sub * SUB + g * 8, 16)]
                            for j in range(8):
                                e = g * 8 + j
                                dl = dl16[j]
                                asrc = hbuf[b, e, pl.ds(FOFF, 16)]
                                al = asrc + adl[dl, pl.ds(0, 16)]
                                al = jnp.where(al >= 0.0, al, 0.2 * al)
                                w = jnp.exp(al)
                                plsc.addupdate(dacc.at[dl, pl.ds(0, 16)], w)
                                for c in range(NCH):
                                    f16 = hbuf[b, e, pl.ds(c * 16, 16)]
                                    plsc.addupdate(
                                        accv.at[dl, pl.ds(c * 16, 16)],
                                        w[c * 16 // CPH] * f16)
                            return 0
                        lax.fori_loop(0, SUB // 8, _grp, 0)
                    return 0

                lax.fori_loop(sc0, sc1, _chunk, 0)

                def _fin(v, _):
                    inv = 1.0 / (dacc[v, pl.ds(0, 16)] + 1e-16)
                    for h in range(H):
                        iv = inv[h]
                        for c in range(CPH // 16):
                            cc = h * CPH + c * 16
                            accv[v, pl.ds(cc, 16)] = accv[v, pl.ds(cc, 16)] * iv
                    return 0
                lax.fori_loop(0, NPT, _fin, 0)
                pltpu.sync_copy(accv.at[pl.ds(0, NPT)],
                                out.at[pl.ds(pl.multiple_of(v0, 8), NPT)])
            return 0

        lax.fori_loop(0, 4, _round, 0)

    return mp


_mp1 = _make_mp(512, HEADS, 640, 512, 512, 32)
_mp2 = _make_mp(128, 1, 128, 64, 64, 128)


def kernel(x, edge_index, batch, W1, att_src1, att_dst1, b1,
           W2, att_src2, att_dst2, b2, Wf1, bf1, Wf2, bf2):
    row = jnp.concatenate([edge_index[0], edge_index[1]]).astype(jnp.int32)
    col = jnp.concatenate([edge_index[1], edge_index[0]]).astype(jnp.int32)
    key = jnp.sort(col * N + row)
    dup = jnp.concatenate([jnp.zeros((1,), jnp.bool_), key[1:] == key[:-1]])
    src = key % N
    dst = key // N
    ew = (src | (dst << 14) | (dup.astype(jnp.int32) << 28)).astype(jnp.int32)
    bounds = jnp.clip(jnp.arange(152, dtype=jnp.int32) * NPT, 0, N)
    tb = jnp.searchsorted(key, bounds * N, side="left").astype(jnp.int32)

    a1s = jnp.zeros((HEADS * H1, 128), jnp.float32)
    a1d = jnp.zeros((HEADS * H1, 128), jnp.float32)
    for h in range(HEADS):
        a1s = a1s.at[h * H1:(h + 1) * H1, h].set(att_src1[h])
        a1d = a1d.at[h * H1:(h + 1) * H1, h].set(att_dst1[h])
    a2s = jnp.zeros((H2, 64), jnp.float32).at[:, 0].set(att_src2[0])
    a2d = jnp.zeros((H2, 128), jnp.float32).at[:, 0].set(att_dst2[0])

    zerod = jnp.zeros((1, D), jnp.float32)

    f1, ad1 = _mm_att(x, W1, a1s, a1d, False, zerod)
    g1 = _mp1(f1, ad1, ew, tb)
    f2, ad2 = _mm_att(g1, W2, a2s, a2d, True, b1.reshape(1, -1))
    g2 = _mp2(f2, ad2, ew, tb)
    return _head(g2[:, :H2], batch.reshape(-1, 1), b2.reshape(1, -1),
                 Wf1, bf1.reshape(1, -1), Wf2, bf2.reshape(1, -1))
